# baseline (speedup 1.0000x reference)
import functools
import math

import jax
import jax.numpy as jnp
from jax import lax
from jax.experimental import pallas as pl
from jax.experimental.pallas import tpu as pltpu
from jax.experimental.pallas import tpu_sc as plsc

N_CLASSES = 256
VEC_LEN = 256
NUM_GROUP = 16
NCPG = N_CLASSES // NUM_GROUP
TARGET_SCALE = 0.06
B0, CH, T0 = 4, VEC_LEN, 256
NTOK = B0 * T0


def _tc_body(x0_ref, e_ref, idx_ref, gs_ref, perp_ref):
    tn = TARGET_SCALE * math.sqrt(CH)
    xf_parts = []
    for b in range(B0):
        xb = x0_ref[b]
        n2 = jnp.sum(xb * xb, axis=0, keepdims=True)
        xf_parts.append(tn * xb / jnp.sqrt(n2))
    xf = jnp.concatenate(xf_parts, axis=0)

    ev = e_ref[...]
    en2 = jnp.sum(ev * ev, axis=1, keepdims=True)
    en = tn * ev / jnp.sqrt(en2)

    ones_row = jnp.ones((1, VEC_LEN), jnp.float32)
    rn2_row = lax.dot_general(
        ones_row, xf * xf, (((1,), (1,)), ((), ())),
        precision=lax.Precision.HIGHEST, preferred_element_type=jnp.float32)
    en2_col = jnp.sum(en * en, axis=1, keepdims=True)

    gt = lax.dot_general(
        en, xf, (((1,), (1,)), ((), ())),
        precision=lax.Precision.HIGHEST,
        preferred_element_type=jnp.float32)
    d2 = jnp.maximum(en2_col + rn2_row - 2.0 * gt, 0.0)
    d = jnp.sqrt(d2)

    gg2 = lax.broadcasted_iota(jnp.int32, (NUM_GROUP, N_CLASSES), 0)
    jj2 = lax.broadcasted_iota(jnp.int32, (NUM_GROUP, N_CLASSES), 1)
    grp2 = jnp.where(jj2 // NCPG == gg2, 1.0, 0.0).astype(jnp.float32)

    dg = lax.dot_general(
        grp2, d, (((1,), (0,)), ((), ())),
        precision=lax.Precision.HIGHEST,
        preferred_element_type=jnp.float32) * (1.0 / NCPG)

    dmin = jnp.min(dg, axis=0, keepdims=True)
    ii = lax.broadcasted_iota(jnp.int32, (NUM_GROUP, NTOK), 0)
    idx = jnp.min(jnp.where(dg == dmin, ii, NUM_GROUP), axis=0, keepdims=True)
    idx_ref[...] = idx

    gs_ref[...] = lax.dot_general(
        grp2, en, (((1,), (0,)), ((), ())),
        precision=lax.Precision.HIGHEST, preferred_element_type=jnp.float32)

    cls = lax.broadcasted_iota(jnp.int32, (N_CLASSES, NTOK), 0) // NCPG
    sel = cls == idx
    p = 1.0 / d
    mp = jnp.where(sel, p, 0.0)
    s = jnp.sum(mp * jnp.log(mp + 1e-10))
    perp_ref[...] = jnp.broadcast_to(jnp.exp(-s), (1, 1))


_tc_call = pl.pallas_call(
    _tc_body,
    out_shape=[
        jax.ShapeDtypeStruct((1, NTOK), jnp.int32),
        jax.ShapeDtypeStruct((NUM_GROUP, VEC_LEN), jnp.float32),
        jax.ShapeDtypeStruct((1, 1), jnp.float32),
    ],
)

_NC, _NS = 2, 16
_NW = _NC * _NS
_BPW = NTOK // _NW
_L = 16


def _sc_gather_body(gs_hbm, idx_hbm, out_hbm, gs_v, idx_v, rows_v, sem):
    wid = lax.axis_index("s") * _NC + lax.axis_index("c")
    base = wid * _BPW

    tbl = pltpu.async_copy(gs_hbm, gs_v, sem)
    pltpu.sync_copy(idx_hbm.at[0, pl.ds(base, _BPW)], idx_v)
    tbl.wait()

    def col_step(_, colv):
        lanes = lax.iota(jnp.int32, _L)
        for tb in range(_BPW // _L):
            idxb = idx_v[pl.ds(tb * _L, _L)]
            val = plsc.load_gather(gs_v, [idxb, colv])
            plsc.store_scatter(rows_v, [lanes + tb * _L, colv], val)
        return colv + 1

    lax.fori_loop(0, VEC_LEN, col_step, jnp.zeros((_L,), jnp.int32))
    pltpu.sync_copy(rows_v, out_hbm.at[pl.ds(base, _BPW)])


@functools.cache
def _sc_gather():
    return pl.kernel(
        _sc_gather_body,
        mesh=plsc.VectorSubcoreMesh(
            core_axis_name="c", subcore_axis_name="s",
            num_cores=_NC, num_subcores=_NS),
        out_type=jax.ShapeDtypeStruct((NTOK, VEC_LEN), jnp.float32),
        scratch_types=[
            pltpu.VMEM((NUM_GROUP, VEC_LEN), jnp.float32),
            pltpu.VMEM((_BPW,), jnp.int32),
            pltpu.VMEM((_BPW, VEC_LEN), jnp.float32),
            pltpu.SemaphoreType.DMA,
        ],
        compiler_params=pltpu.CompilerParams(needs_layout_passes=False),
    )


def kernel(x0, embedding0):
    idx, gs, perp = _tc_call(x0, embedding0)
    quant = _sc_gather()(gs, idx)
    return quant, jnp.reshape(perp, ())


# speedup vs baseline: 1.2861x; 1.2861x over previous
import functools
import math

import jax
import jax.numpy as jnp
from jax import lax
from jax.experimental import pallas as pl
from jax.experimental.pallas import tpu as pltpu
from jax.experimental.pallas import tpu_sc as plsc

N_CLASSES = 256
VEC_LEN = 256
NUM_GROUP = 16
NCPG = N_CLASSES // NUM_GROUP
TARGET_SCALE = 0.06
B0, CH, T0 = 4, VEC_LEN, 256
NTOK = B0 * T0


def _tc_body(x0_ref, e_ref, idx_ref, gs_ref, perp_ref):
    tn = TARGET_SCALE * math.sqrt(CH)
    xf_parts = []
    for b in range(B0):
        xb = x0_ref[b]
        n2 = jnp.sum(xb * xb, axis=0, keepdims=True)
        xf_parts.append(tn * xb / jnp.sqrt(n2))
    xf = jnp.concatenate(xf_parts, axis=0)

    ev = e_ref[...]
    en2 = jnp.sum(ev * ev, axis=1, keepdims=True)
    en = tn * ev / jnp.sqrt(en2)

    ones_row = jnp.ones((1, VEC_LEN), jnp.float32)
    rn2_row = lax.dot_general(
        ones_row, xf * xf, (((1,), (1,)), ((), ())),
        precision=lax.Precision.HIGHEST, preferred_element_type=jnp.float32)
    en2_col = jnp.sum(en * en, axis=1, keepdims=True)

    gt = lax.dot_general(
        en, xf, (((1,), (1,)), ((), ())),
        precision=lax.Precision.HIGHEST,
        preferred_element_type=jnp.float32)
    d2 = jnp.maximum(en2_col + rn2_row - 2.0 * gt, 0.0)
    d = jnp.sqrt(d2)

    gg2 = lax.broadcasted_iota(jnp.int32, (NUM_GROUP, N_CLASSES), 0)
    jj2 = lax.broadcasted_iota(jnp.int32, (NUM_GROUP, N_CLASSES), 1)
    grp2 = jnp.where(jj2 // NCPG == gg2, 1.0, 0.0).astype(jnp.float32)

    dg = lax.dot_general(
        grp2, d, (((1,), (0,)), ((), ())),
        precision=lax.Precision.HIGHEST,
        preferred_element_type=jnp.float32) * (1.0 / NCPG)

    dmin = jnp.min(dg, axis=0, keepdims=True)
    ii = lax.broadcasted_iota(jnp.int32, (NUM_GROUP, NTOK), 0)
    idx = jnp.min(jnp.where(dg == dmin, ii, NUM_GROUP), axis=0, keepdims=True)
    idx_ref[...] = idx

    gs_ref[...] = lax.dot_general(
        grp2, en, (((1,), (0,)), ((), ())),
        precision=lax.Precision.HIGHEST, preferred_element_type=jnp.float32)

    cls = lax.broadcasted_iota(jnp.int32, (N_CLASSES, NTOK), 0) // NCPG
    sel = cls == idx
    p = 1.0 / d
    mp = jnp.where(sel, p, 0.0)
    s = jnp.sum(mp * jnp.log(mp + 1e-10))
    perp_ref[...] = jnp.broadcast_to(jnp.exp(-s), (1, 1))


_tc_call = pl.pallas_call(
    _tc_body,
    out_shape=[
        jax.ShapeDtypeStruct((1, NTOK), jnp.int32),
        jax.ShapeDtypeStruct((NUM_GROUP, VEC_LEN), jnp.float32),
        jax.ShapeDtypeStruct((1, 1), jnp.float32),
    ],
)

_NC, _NS = 2, 16
_NW = _NC * _NS
_BPW = NTOK // _NW
_L = 16


def _sc_gather_body(gs_hbm, idx_hbm, out_hbm, gs_v, idx_v, rows_v, sem):
    wid = lax.axis_index("s") * _NC + lax.axis_index("c")
    base = wid * _BPW

    tbl = pltpu.async_copy(gs_hbm, gs_v, sem)
    pltpu.sync_copy(idx_hbm.at[0, pl.ds(base, _BPW)], idx_v)
    tbl.wait()

    lanes = lax.iota(jnp.int32, _L)
    idx_blocks = [idx_v[pl.ds(tb * _L, _L)] for tb in range(_BPW // _L)]

    @plsc.parallel_loop(0, VEC_LEN, unroll=8)
    def _col_step(j):
        colv = jnp.full((_L,), j, jnp.int32)
        for tb in range(_BPW // _L):
            val = plsc.load_gather(gs_v, [idx_blocks[tb], colv])
            plsc.store_scatter(rows_v, [lanes + tb * _L, colv], val)

    pltpu.sync_copy(rows_v, out_hbm.at[pl.ds(base, _BPW)])


@functools.cache
def _sc_gather():
    return pl.kernel(
        _sc_gather_body,
        mesh=plsc.VectorSubcoreMesh(
            core_axis_name="c", subcore_axis_name="s",
            num_cores=_NC, num_subcores=_NS),
        out_type=jax.ShapeDtypeStruct((NTOK, VEC_LEN), jnp.float32),
        scratch_types=[
            pltpu.VMEM((NUM_GROUP, VEC_LEN), jnp.float32),
            pltpu.VMEM((_BPW,), jnp.int32),
            pltpu.VMEM((_BPW, VEC_LEN), jnp.float32),
            pltpu.SemaphoreType.DMA,
        ],
        compiler_params=pltpu.CompilerParams(needs_layout_passes=False),
    )


def kernel(x0, embedding0):
    idx, gs, perp = _tc_call(x0, embedding0)
    quant = _sc_gather()(gs, idx)
    return quant, jnp.reshape(perp, ())


# speedup vs baseline: 1.3343x; 1.0375x over previous
import functools
import math

import jax
import jax.numpy as jnp
from jax import lax
from jax.experimental import pallas as pl
from jax.experimental.pallas import tpu as pltpu
from jax.experimental.pallas import tpu_sc as plsc

N_CLASSES = 256
VEC_LEN = 256
NUM_GROUP = 16
NCPG = N_CLASSES // NUM_GROUP
TARGET_SCALE = 0.06
B0, CH, T0 = 4, VEC_LEN, 256
NTOK = B0 * T0


def _tc_body(x0_ref, e_ref, idx_ref, gs_ref, perp_ref):
    tn = TARGET_SCALE * math.sqrt(CH)
    xf_parts = []
    for b in range(B0):
        xb = x0_ref[b]
        n2 = jnp.sum(xb * xb, axis=0, keepdims=True)
        xf_parts.append(tn * xb / jnp.sqrt(n2))
    xf = jnp.concatenate(xf_parts, axis=0)

    ev = e_ref[...]
    en2 = jnp.sum(ev * ev, axis=1, keepdims=True)
    en = tn * ev / jnp.sqrt(en2)

    ones_row = jnp.ones((1, VEC_LEN), jnp.float32)
    rn2_row = lax.dot_general(
        ones_row, xf * xf, (((1,), (1,)), ((), ())),
        precision=lax.Precision.HIGHEST, preferred_element_type=jnp.float32)
    en2_col = jnp.sum(en * en, axis=1, keepdims=True)

    gt = lax.dot_general(
        en, xf, (((1,), (1,)), ((), ())),
        precision=lax.Precision.HIGHEST,
        preferred_element_type=jnp.float32)
    d2 = jnp.maximum(en2_col + rn2_row - 2.0 * gt, 0.0)
    d = jnp.sqrt(d2)

    gg2 = lax.broadcasted_iota(jnp.int32, (NUM_GROUP, N_CLASSES), 0)
    jj2 = lax.broadcasted_iota(jnp.int32, (NUM_GROUP, N_CLASSES), 1)
    grp2 = jnp.where(jj2 // NCPG == gg2, 1.0, 0.0).astype(jnp.float32)

    dg = lax.dot_general(
        grp2, d, (((1,), (0,)), ((), ())),
        precision=lax.Precision.HIGHEST,
        preferred_element_type=jnp.float32) * (1.0 / NCPG)

    dmin = jnp.min(dg, axis=0, keepdims=True)
    ii = lax.broadcasted_iota(jnp.int32, (NUM_GROUP, NTOK), 0)
    idx = jnp.min(jnp.where(dg == dmin, ii, NUM_GROUP), axis=0, keepdims=True)
    idx_ref[...] = idx

    gs_ref[...] = lax.dot_general(
        grp2, en, (((1,), (0,)), ((), ())),
        precision=lax.Precision.HIGHEST, preferred_element_type=jnp.float32)

    cls = lax.broadcasted_iota(jnp.int32, (N_CLASSES, NTOK), 0) // NCPG
    sel = cls == idx
    p = 1.0 / d
    mp = jnp.where(sel, p, 0.0)
    s = jnp.sum(mp * jnp.log(mp + 1e-10))
    perp_ref[...] = jnp.broadcast_to(jnp.exp(-s), (1, 1))


_tc_call = pl.pallas_call(
    _tc_body,
    out_shape=[
        jax.ShapeDtypeStruct((1, NTOK), jnp.int32),
        jax.ShapeDtypeStruct((NUM_GROUP, VEC_LEN), jnp.float32),
        jax.ShapeDtypeStruct((1, 1), jnp.float32),
    ],
)

_NC, _NS = 2, 16
_NW = _NC * _NS
_BPW = NTOK // _NW
_L = 16


def _sc_gather_body(gs_hbm, idx_hbm, out_hbm, idx_v, rows_v, s0, s1, s2, s3):
    wid = lax.axis_index("s") * _NC + lax.axis_index("c")
    base = wid * _BPW
    half = _BPW // 2

    pltpu.sync_copy(idx_hbm.at[0, pl.ds(base, _BPW)], idx_v)
    g0 = pltpu.async_copy(
        gs_hbm.at[idx_v.at[pl.ds(0, half)]], rows_v.at[pl.ds(0, half)], s0)
    g1 = pltpu.async_copy(
        gs_hbm.at[idx_v.at[pl.ds(half, half)]], rows_v.at[pl.ds(half, half)],
        s1)
    g0.wait()
    o0 = pltpu.async_copy(
        rows_v.at[pl.ds(0, half)], out_hbm.at[pl.ds(base, half)], s2)
    g1.wait()
    o1 = pltpu.async_copy(
        rows_v.at[pl.ds(half, half)], out_hbm.at[pl.ds(base + half, half)],
        s3)
    o0.wait()
    o1.wait()


@functools.cache
def _sc_gather():
    return pl.kernel(
        _sc_gather_body,
        mesh=plsc.VectorSubcoreMesh(
            core_axis_name="c", subcore_axis_name="s",
            num_cores=_NC, num_subcores=_NS),
        out_type=jax.ShapeDtypeStruct((NTOK, VEC_LEN), jnp.float32),
        scratch_types=[
            pltpu.VMEM((_BPW,), jnp.int32),
            pltpu.VMEM((_BPW, VEC_LEN), jnp.float32),
            pltpu.SemaphoreType.DMA,
            pltpu.SemaphoreType.DMA,
            pltpu.SemaphoreType.DMA,
            pltpu.SemaphoreType.DMA,
        ],
        compiler_params=pltpu.CompilerParams(needs_layout_passes=False),
    )


def kernel(x0, embedding0):
    idx, gs, perp = _tc_call(x0, embedding0)
    quant = _sc_gather()(gs, idx)
    return quant, jnp.reshape(perp, ())
